# baseline (speedup 1.0000x reference)
import jax
import jax.numpy as jnp
from jax.experimental import pallas as pl
from jax.experimental.pallas import tpu as pltpu

_INTERP_H = 14
_INTERP_W = 14
_G = 16


def _roi_matrix(rois_ref, m, h, w, ih, iw):
    p = ih * iw
    q = h * w
    x1 = rois_ref[m, 0] * (w - 1)
    y1 = rois_ref[m, 1] * (h - 1)
    x2 = rois_ref[m, 2] * (w - 1)
    y2 = rois_ref[m, 3] * (h - 1)

    r = jax.lax.broadcasted_iota(jnp.int32, (p, 1), 0)
    i = (r // iw).astype(jnp.float32)
    j = (r % iw).astype(jnp.float32)
    ys = jnp.clip(y1 + (y2 - y1) * (i * (1.0 / (ih - 1))), 0.0, h - 1.0)
    xs = jnp.clip(x1 + (x2 - x1) * (j * (1.0 / (iw - 1))), 0.0, w - 1.0)

    c = jax.lax.broadcasted_iota(jnp.int32, (1, q), 1)
    y = (c // w).astype(jnp.float32)
    x = (c % w).astype(jnp.float32)

    ay = jnp.maximum(1.0 - jnp.abs(ys - y), 0.0)
    ax = jnp.maximum(1.0 - jnp.abs(xs - x), 0.0)
    return (ay * ax).astype(jnp.bfloat16)


def _interp_kernel(idx_ref, rois_ref, in_ref, in_hbm, out_ref,
                   bounce, lb, sems):
    g = pl.program_id(0)
    h, w = 14, 14
    base = g * _G

    for k in range(_G):
        m = idx_ref[base + k]

        @pl.when(m != base + k)
        def _start(k=k, m=m):
            pltpu.make_async_copy(in_hbm.at[m], bounce.at[k],
                                  sems.at[k]).start()

    for k in range(_G):
        m = idx_ref[base + k]
        matched = m == base + k

        @pl.when(matched)
        def _fast(k=k):
            lb[...] = in_ref[k].astype(jnp.bfloat16)

        @pl.when(jnp.logical_not(matched))
        def _slow(k=k, m=m):
            pltpu.make_async_copy(in_hbm.at[m], bounce.at[k],
                                  sems.at[k]).wait()
            lb[...] = bounce[k].astype(jnp.bfloat16)

        mmat = _roi_matrix(rois_ref, m, h, w, _INTERP_H, _INTERP_W)
        out_ref[k] = jax.lax.dot_general(
            lb[...], mmat,
            dimension_numbers=(((1,), (1,)), ((), ())),
            preferred_element_type=jnp.float32,
        )


def kernel(input, rois):
    n, ch, h, w = input.shape
    q = h * w
    p = _INTERP_H * _INTERP_W
    mask = ~((rois[:, 0] == 0) & (rois[:, 2] == 0))
    idx = jnp.nonzero(mask, size=n, fill_value=0)[0].astype(jnp.int32)
    inp_flat = input.reshape(n, ch, q)

    grid_spec = pltpu.PrefetchScalarGridSpec(
        num_scalar_prefetch=2,
        grid=(n // _G,),
        in_specs=[
            pl.BlockSpec((_G, ch, q), lambda g, idx_ref, rois_ref: (g, 0, 0)),
            pl.BlockSpec(memory_space=pltpu.MemorySpace.HBM),
        ],
        out_specs=pl.BlockSpec((_G, ch, p),
                               lambda g, idx_ref, rois_ref: (g, 0, 0)),
        scratch_shapes=[
            pltpu.MemorySpace.VMEM((_G, ch, q), jnp.float32),
            pltpu.MemorySpace.VMEM((ch, q), jnp.bfloat16),
            pltpu.SemaphoreType.DMA((_G,)),
        ],
    )
    out = pl.pallas_call(
        _interp_kernel,
        grid_spec=grid_spec,
        out_shape=jax.ShapeDtypeStruct((n, ch, p), jnp.float32),
        compiler_params=pltpu.CompilerParams(vmem_limit_bytes=100 * 1024 * 1024),
    )(idx, rois, inp_flat, inp_flat)
    return out.reshape(n, ch, _INTERP_H, _INTERP_W)


# speedup vs baseline: 1.3717x; 1.3717x over previous
import jax
import jax.numpy as jnp
from jax.experimental import pallas as pl
from jax.experimental.pallas import tpu as pltpu

_INTERP_H = 14
_INTERP_W = 14
_G = 16


def _roi_matrix(rois_ref, m, h, w, ih, iw):
    p = ih * iw
    q = h * w
    x1 = rois_ref[m, 0] * (w - 1)
    y1 = rois_ref[m, 1] * (h - 1)
    x2 = rois_ref[m, 2] * (w - 1)
    y2 = rois_ref[m, 3] * (h - 1)

    r = jax.lax.broadcasted_iota(jnp.int32, (p, 1), 0)
    i = (r // iw).astype(jnp.float32)
    j = (r % iw).astype(jnp.float32)
    ys = jnp.clip(y1 + (y2 - y1) * (i * (1.0 / (ih - 1))), 0.0, h - 1.0)
    xs = jnp.clip(x1 + (x2 - x1) * (j * (1.0 / (iw - 1))), 0.0, w - 1.0)

    c = jax.lax.broadcasted_iota(jnp.int32, (1, q), 1)
    y = (c // w).astype(jnp.float32)
    x = (c % w).astype(jnp.float32)

    ay = jnp.maximum(1.0 - jnp.abs(ys - y), 0.0)
    ax = jnp.maximum(1.0 - jnp.abs(xs - x), 0.0)
    return (ay * ax).astype(jnp.bfloat16)


_NCHUNK = 16


def _interp_kernel(idx_ref, rois_ref, need_ref, prev_ref, in_hbm, out_ref,
                   in_vmem, sems):
    g = pl.program_id(0)
    h, w = 14, 14
    n = in_vmem.shape[0]
    rows = n // _NCHUNK

    @pl.when(g == 0)
    def _stage():
        for c in range(_NCHUNK):
            sl = pl.ds(c * rows, rows)
            pltpu.make_async_copy(in_hbm.at[sl], in_vmem.at[sl],
                                  sems.at[c]).start()

    for c in range(_NCHUNK):
        @pl.when((prev_ref[g] < c) & (c <= need_ref[g]))
        def _wait(c=c):
            sl = pl.ds(c * rows, rows)
            pltpu.make_async_copy(in_hbm.at[sl], in_vmem.at[sl],
                                  sems.at[c]).wait()

    for k in range(_G):
        m = idx_ref[g * _G + k]
        mmat = _roi_matrix(rois_ref, m, h, w, _INTERP_H, _INTERP_W)
        out_ref[k] = jax.lax.dot_general(
            in_vmem[m], mmat,
            dimension_numbers=(((0,), (1,)), ((), ())),
            preferred_element_type=jnp.float32,
        )


def kernel(input, rois):
    n, ch, h, w = input.shape
    q = h * w
    p = _INTERP_H * _INTERP_W
    mask = ~((rois[:, 0] == 0) & (rois[:, 2] == 0))
    idx = jnp.nonzero(mask, size=n, fill_value=0)[0].astype(jnp.int32)
    inp_t = jnp.swapaxes(input.reshape(n, ch, q), 1, 2).astype(jnp.bfloat16)

    rows_per_chunk = n // _NCHUNK
    bmax = jnp.max(idx.reshape(n // _G, _G), axis=1)
    need = (jax.lax.cummax(bmax) // rows_per_chunk).astype(jnp.int32)
    prev = jnp.concatenate([jnp.full((1,), -1, jnp.int32), need[:-1]])

    grid_spec = pltpu.PrefetchScalarGridSpec(
        num_scalar_prefetch=4,
        grid=(n // _G,),
        in_specs=[pl.BlockSpec(memory_space=pltpu.MemorySpace.HBM)],
        out_specs=pl.BlockSpec((_G, ch, p),
                               lambda g, idx_ref, rois_ref, need_ref, prev_ref:
                               (g, 0, 0)),
        scratch_shapes=[
            pltpu.MemorySpace.VMEM((n, q, ch), jnp.bfloat16),
            pltpu.SemaphoreType.DMA((_NCHUNK,)),
        ],
    )
    out = pl.pallas_call(
        _interp_kernel,
        grid_spec=grid_spec,
        out_shape=jax.ShapeDtypeStruct((n, ch, p), jnp.float32),
        compiler_params=pltpu.CompilerParams(vmem_limit_bytes=100 * 1024 * 1024),
    )(idx, rois, need, prev, inp_t)
    return out.reshape(n, ch, _INTERP_H, _INTERP_W)


# speedup vs baseline: 1.3723x; 1.0005x over previous
import jax
import jax.numpy as jnp
from jax.experimental import pallas as pl
from jax.experimental.pallas import tpu as pltpu

_INTERP_H = 14
_INTERP_W = 14
_G = 16


def _roi_matrix(rois_ref, m, h, w, ih, iw):
    p = ih * iw
    q = h * w
    x1 = rois_ref[m, 0] * (w - 1)
    y1 = rois_ref[m, 1] * (h - 1)
    x2 = rois_ref[m, 2] * (w - 1)
    y2 = rois_ref[m, 3] * (h - 1)

    r = jax.lax.broadcasted_iota(jnp.int32, (p, 1), 0)
    i = (r // iw).astype(jnp.float32)
    j = (r % iw).astype(jnp.float32)
    ys = jnp.clip(y1 + (y2 - y1) * (i * (1.0 / (ih - 1))), 0.0, h - 1.0)
    xs = jnp.clip(x1 + (x2 - x1) * (j * (1.0 / (iw - 1))), 0.0, w - 1.0)

    c = jax.lax.broadcasted_iota(jnp.int32, (1, q), 1)
    y = (c // w).astype(jnp.float32)
    x = (c % w).astype(jnp.float32)

    ay = jnp.maximum(1.0 - jnp.abs(ys - y), 0.0)
    ax = jnp.maximum(1.0 - jnp.abs(xs - x), 0.0)
    return (ay * ax).astype(jnp.bfloat16)


_NCHUNK = 8


def _interp_kernel(idx_ref, rois_ref, need_ref, prev_ref, in_hbm, out_ref,
                   in_vmem, sems):
    g = pl.program_id(0)
    h, w = 14, 14
    n = in_vmem.shape[0]
    rows = n // _NCHUNK

    @pl.when(g == 0)
    def _stage():
        for c in range(_NCHUNK):
            sl = pl.ds(c * rows, rows)
            pltpu.make_async_copy(in_hbm.at[sl], in_vmem.at[sl],
                                  sems.at[c]).start()

    for c in range(_NCHUNK):
        @pl.when((prev_ref[g] < c) & (c <= need_ref[g]))
        def _wait(c=c):
            sl = pl.ds(c * rows, rows)
            pltpu.make_async_copy(in_hbm.at[sl], in_vmem.at[sl],
                                  sems.at[c]).wait()

    for k in range(_G):
        m = idx_ref[g * _G + k]
        mmat = _roi_matrix(rois_ref, m, h, w, _INTERP_H, _INTERP_W)
        out_ref[k] = jax.lax.dot_general(
            in_vmem[m], mmat,
            dimension_numbers=(((0,), (1,)), ((), ())),
            preferred_element_type=jnp.float32,
        )


def kernel(input, rois):
    n, ch, h, w = input.shape
    q = h * w
    p = _INTERP_H * _INTERP_W
    mask = ~((rois[:, 0] == 0) & (rois[:, 2] == 0))
    idx = jnp.nonzero(mask, size=n, fill_value=0)[0].astype(jnp.int32)
    inp_t = jnp.swapaxes(input.reshape(n, ch, q), 1, 2).astype(jnp.bfloat16)

    rows_per_chunk = n // _NCHUNK
    bmax = jnp.max(idx.reshape(n // _G, _G), axis=1)
    need = (jax.lax.cummax(bmax) // rows_per_chunk).astype(jnp.int32)
    prev = jnp.concatenate([jnp.full((1,), -1, jnp.int32), need[:-1]])

    grid_spec = pltpu.PrefetchScalarGridSpec(
        num_scalar_prefetch=4,
        grid=(n // _G,),
        in_specs=[pl.BlockSpec(memory_space=pltpu.MemorySpace.HBM)],
        out_specs=pl.BlockSpec((_G, ch, p),
                               lambda g, idx_ref, rois_ref, need_ref, prev_ref:
                               (g, 0, 0)),
        scratch_shapes=[
            pltpu.MemorySpace.VMEM((n, q, ch), jnp.bfloat16),
            pltpu.SemaphoreType.DMA((_NCHUNK,)),
        ],
    )
    out = pl.pallas_call(
        _interp_kernel,
        grid_spec=grid_spec,
        out_shape=jax.ShapeDtypeStruct((n, ch, p), jnp.float32),
        compiler_params=pltpu.CompilerParams(vmem_limit_bytes=100 * 1024 * 1024),
    )(idx, rois, need, prev, inp_t)
    return out.reshape(n, ch, _INTERP_H, _INTERP_W)


# speedup vs baseline: 1.4051x; 1.0239x over previous
import functools

import jax
import jax.numpy as jnp
from jax import lax
from jax.experimental import pallas as pl
from jax.experimental.pallas import tpu as pltpu
from jax.experimental.pallas import tpu_sc as plsc

_INTERP_H = 14
_INTERP_W = 14
_G = 16


def _roi_matrix(rois_ref, m, h, w, ih, iw):
    p = ih * iw
    q = h * w
    x1 = rois_ref[m, 0] * (w - 1)
    y1 = rois_ref[m, 1] * (h - 1)
    x2 = rois_ref[m, 2] * (w - 1)
    y2 = rois_ref[m, 3] * (h - 1)

    r = jax.lax.broadcasted_iota(jnp.int32, (p, 1), 0)
    i = (r // iw).astype(jnp.float32)
    j = (r % iw).astype(jnp.float32)
    ys = jnp.clip(y1 + (y2 - y1) * (i * (1.0 / (ih - 1))), 0.0, h - 1.0)
    xs = jnp.clip(x1 + (x2 - x1) * (j * (1.0 / (iw - 1))), 0.0, w - 1.0)

    c = jax.lax.broadcasted_iota(jnp.int32, (1, q), 1)
    y = (c // w).astype(jnp.float32)
    x = (c % w).astype(jnp.float32)

    ay = jnp.maximum(1.0 - jnp.abs(ys - y), 0.0)
    ax = jnp.maximum(1.0 - jnp.abs(xs - x), 0.0)
    return (ay * ax).astype(jnp.bfloat16)


_NCHUNK = 8


def _interp_kernel(idx_ref, rois_ref, need_ref, prev_ref, in_hbm, out_ref,
                   in_vmem, sems):
    g = pl.program_id(0)
    h, w = 14, 14
    n = in_vmem.shape[0]
    rows = n // _NCHUNK

    @pl.when(g == 0)
    def _stage():
        for c in range(_NCHUNK):
            sl = pl.ds(c * rows, rows)
            pltpu.make_async_copy(in_hbm.at[sl], in_vmem.at[sl],
                                  sems.at[c]).start()

    for c in range(_NCHUNK):
        @pl.when((prev_ref[g] < c) & (c <= need_ref[g]))
        def _wait(c=c):
            sl = pl.ds(c * rows, rows)
            pltpu.make_async_copy(in_hbm.at[sl], in_vmem.at[sl],
                                  sems.at[c]).wait()

    for k in range(_G):
        m = idx_ref[g * _G + k]
        mmat = _roi_matrix(rois_ref, m, h, w, _INTERP_H, _INTERP_W)
        out_ref[k] = jax.lax.dot_general(
            in_vmem[m], mmat,
            dimension_numbers=(((0,), (1,)), ((), ())),
            preferred_element_type=jnp.float32,
        )


def _make_compact_idx(n):
    @functools.partial(
        pl.kernel,
        mesh=plsc.ScalarSubcoreMesh(
            axis_name="c", num_cores=plsc.get_sparse_core_info().num_cores),
        out_type=jax.ShapeDtypeStruct((n,), jnp.int32),
        scratch_types=[
            pltpu.SMEM((n * 4,), jnp.float32),
            pltpu.SMEM((n,), jnp.int32),
            pltpu.SemaphoreType.DMA,
        ],
    )
    def _compact(rois_hbm, idx_hbm, rois_s, idx_s, sem):
        @pl.when(lax.axis_index("c") == 0)
        def _():
            pltpu.async_copy(rois_hbm, rois_s, sem).wait()

            def fill(i, carry):
                idx_s[i] = 0
                return carry

            lax.fori_loop(0, n, fill, 0)

            def body(i, count):
                keep = jnp.logical_not(
                    (rois_s[i * 4] == 0.0) & (rois_s[i * 4 + 2] == 0.0))

                @pl.when(keep)
                def _store():
                    idx_s[count] = i

                return count + keep.astype(jnp.int32)

            lax.fori_loop(0, n, body, jnp.int32(0))
            pltpu.async_copy(idx_s, idx_hbm, sem).wait()

    return _compact


def kernel(input, rois):
    n, ch, h, w = input.shape
    q = h * w
    p = _INTERP_H * _INTERP_W
    idx = _make_compact_idx(n)(rois.reshape(n * 4))
    inp_t = jnp.swapaxes(input.reshape(n, ch, q), 1, 2).astype(jnp.bfloat16)

    rows_per_chunk = n // _NCHUNK
    bmax = jnp.max(idx.reshape(n // _G, _G), axis=1)
    need = (jax.lax.cummax(bmax) // rows_per_chunk).astype(jnp.int32)
    prev = jnp.concatenate([jnp.full((1,), -1, jnp.int32), need[:-1]])

    grid_spec = pltpu.PrefetchScalarGridSpec(
        num_scalar_prefetch=4,
        grid=(n // _G,),
        in_specs=[pl.BlockSpec(memory_space=pltpu.MemorySpace.HBM)],
        out_specs=pl.BlockSpec((_G, ch, p),
                               lambda g, idx_ref, rois_ref, need_ref, prev_ref:
                               (g, 0, 0)),
        scratch_shapes=[
            pltpu.MemorySpace.VMEM((n, q, ch), jnp.bfloat16),
            pltpu.SemaphoreType.DMA((_NCHUNK,)),
        ],
    )
    out = pl.pallas_call(
        _interp_kernel,
        grid_spec=grid_spec,
        out_shape=jax.ShapeDtypeStruct((n, ch, p), jnp.float32),
        compiler_params=pltpu.CompilerParams(vmem_limit_bytes=100 * 1024 * 1024),
    )(idx, rois, need, prev, inp_t)
    return out.reshape(n, ch, _INTERP_H, _INTERP_W)
